# baseline (speedup 1.0000x reference)
import functools

import jax
import jax.numpy as jnp
from jax import lax
from jax.experimental import pallas as pl
from jax.experimental.pallas import tpu as pltpu
from jax.experimental.pallas import tpu_sc as plsc

B = 2
H, W = 64, 1024
h, w = 32, 512
C = 128
KS0, KS1 = 10, 20
K = 3
DIST2 = 100.0 * 100.0
NQ = B * H * W
NQ3 = K * NQ
UP_H = 84
UP_W = 1064
ROWS_PER_STEP = 8
N_STRIPS = H // ROWS_PER_STEP


def _knn_body(q_ref, up_ref, idx_ref, wt_ref):
    b = pl.program_id(0)
    s = pl.program_id(1)
    shp = (ROWS_PER_STEP, W)
    qx = q_ref[0, 0]
    qy = q_ref[0, 1]
    qz = q_ref[0, 2]

    ri = lax.broadcasted_iota(jnp.int32, shp, 0)
    jj = lax.broadcasted_iota(jnp.int32, shp, 1)
    inf = jnp.full(shp, jnp.inf, jnp.float32)
    zero_i = jnp.zeros(shp, jnp.int32)

    def body(dh, st):
        b0, b1, b2, i0, i1, i2 = st
        u = 8 * s + 2 * dh + ri
        mh = (u >= 10) & (u <= 73)
        cx = up_ref[0, dh, 0]
        cy = up_ref[0, dh, 1]
        cz = up_ref[0, dh, 2]
        for dw in range(KS1):
            t0 = 2 * dw
            dx = cx[:, t0:t0 + W] - qx
            dy = cy[:, t0:t0 + W] - qy
            dz = cz[:, t0:t0 + W] - qz
            d2 = dx * dx + dy * dy + dz * dz
            mw = (jj >= 20 - t0) & (jj <= 1043 - t0)
            d2 = jnp.where(mh & mw, d2, inf)
            ci = jnp.full(shp, dh * KS1 + dw, jnp.int32)
            m2 = d2 < b2
            nb2 = jnp.where(m2, d2, b2)
            ni2 = jnp.where(m2, ci, i2)
            m1 = nb2 < b1
            b2 = jnp.where(m1, b1, nb2)
            i2 = jnp.where(m1, i1, ni2)
            nb1 = jnp.where(m1, nb2, b1)
            ni1 = jnp.where(m1, ni2, i1)
            m0 = nb1 < b0
            b1 = jnp.where(m0, b0, nb1)
            i1 = jnp.where(m0, i0, ni1)
            b0 = jnp.where(m0, nb1, b0)
            i0 = jnp.where(m0, ni1, i0)
        return b0, b1, b2, i0, i1, i2

    st = (inf, inf, inf, zero_i, zero_i, zero_i)
    for dh in range(KS0):
        st = body(dh, st)
    b0, b1, b2, i0, i1, i2 = st

    qn = jnp.sqrt(qx * qx + qy * qy + qz * qz)
    irow_half = (8 * s + ri) >> 1
    jcol_half = jj >> 1
    base_flat = irow_half * w + jcol_half + b * (h * w)

    dists = []
    valids = []
    for bk in (b0, b1, b2):
        vk = bk < DIST2
        dk = jnp.where(vk, jnp.sqrt(bk), qn)
        dk = jnp.maximum(dk, 1e-8)
        dists.append(dk)
        valids.append(vk)
    w0 = 1.0 / dists[0]
    w1 = 1.0 / dists[1]
    w2 = 1.0 / dists[2]
    ws = w0 + w1 + w2
    for k, (ik, wk, vk) in enumerate(((i0, w0, valids[0]),
                                      (i1, w1, valids[1]),
                                      (i2, w2, valids[2]))):
        dh_k = ik // KS1
        dw_k = ik - dh_k * KS1
        flat = base_flat + (dh_k - 5) * w + (dw_k - 10)
        flat = jnp.clip(flat, 0, B * h * w - 1)
        idx_ref[k] = flat
        wt_ref[k] = (wk / ws) * vk.astype(jnp.float32)


def _knn_search(query_xyz, up):
    grid = (B, N_STRIPS)
    out_shape = [
        jax.ShapeDtypeStruct((K, B * H, W), jnp.int32),
        jax.ShapeDtypeStruct((K, B * H, W), jnp.float32),
    ]
    return pl.pallas_call(
        _knn_body,
        grid=grid,
        in_specs=[
            pl.BlockSpec((1, 3, ROWS_PER_STEP, W), lambda b, s: (b, 0, s, 0)),
            pl.BlockSpec((1, KS0, 3, ROWS_PER_STEP, UP_W),
                         lambda b, s: (b, 0, 0, s, 0)),
        ],
        out_specs=[
            pl.BlockSpec((K, ROWS_PER_STEP, W),
                         lambda b, s: (0, b * N_STRIPS + s, 0)),
            pl.BlockSpec((K, ROWS_PER_STEP, W),
                         lambda b, s: (0, b * N_STRIPS + s, 0)),
        ],
        out_shape=out_shape,
        compiler_params=pltpu.CompilerParams(
            dimension_semantics=("parallel", "parallel")),
    )(query_xyz, up)


NW = 32
BH = B * H
RPW = BH // NW
CQ = 16
BLKS = W // CQ
N_CH = RPW * BLKS
PER_W = K * RPW * W
RB = K * CQ


WX = K * CQ * CQ


def _sc_fused(feat_t, idx2, wtx):
    mesh = plsc.VectorSubcoreMesh(core_axis_name="c", subcore_axis_name="s")

    @functools.partial(
        pl.kernel,
        mesh=mesh,
        out_type=jax.ShapeDtypeStruct((NQ, C), jnp.float32),
        scratch_types=[
            pltpu.VMEM((PER_W,), jnp.int32),
            pltpu.VMEM((2, WX), jnp.float32),
            pltpu.VMEM((2, RB, C), jnp.float32),
            pltpu.VMEM((2, CQ, C), jnp.float32),
            pltpu.SemaphoreType.DMA,
            pltpu.SemaphoreType.DMA,
            pltpu.SemaphoreType.DMA,
            pltpu.SemaphoreType.DMA,
            pltpu.SemaphoreType.DMA,
            pltpu.SemaphoreType.DMA,
            pltpu.SemaphoreType.DMA,
        ],
    )
    def fused_kernel(feat_hbm, idx_hbm, wtx_hbm, out_hbm, idx_v, wbuf,
                     rbuf, obuf, isem, g0, g1, t0, t1, w0s, w1s):
        wid = lax.axis_index("s") * 2 + lax.axis_index("c")
        row0 = wid * RPW
        for k in range(K):
            pltpu.async_copy(idx_hbm.at[pl.ds(k * NQ + row0 * W, RPW * W)],
                             idx_v.at[pl.ds(k * RPW * W, RPW * W)],
                             isem).wait()

        gsem = (g0, g1)
        tsem = (t0, t1)
        wsem = (w0s, w1s)

        def gq0(ci):
            return (row0 + ci // BLKS) * W + (ci % BLKS) * CQ

        def start_gather(ci, buf):
            off = ci * CQ
            for k in range(K):
                pltpu.async_copy(
                    feat_hbm.at[idx_v.at[pl.ds(k * RPW * W + off, CQ)]],
                    rbuf.at[buf, pl.ds(k * CQ, CQ)], gsem[buf])
            pltpu.async_copy(wtx_hbm.at[pl.ds(gq0(ci) * K * CQ, WX)],
                             wbuf.at[buf], tsem[buf])

        def drain_gather(buf):
            pltpu.make_async_copy(feat_hbm.at[pl.ds(0, RB)], rbuf.at[buf],
                                  gsem[buf]).wait()
            pltpu.make_async_copy(wtx_hbm.at[pl.ds(0, WX)], wbuf.at[buf],
                                  tsem[buf]).wait()

        def compute(buf):
            for q in range(CQ):
                wv = [wbuf[buf, pl.ds((q * K + k) * CQ, CQ)]
                      for k in range(K)]
                for v in range(C // CQ):
                    s = pl.ds(v * CQ, CQ)
                    acc = rbuf[buf, q, s] * wv[0]
                    acc += rbuf[buf, CQ + q, s] * wv[1]
                    acc += rbuf[buf, 2 * CQ + q, s] * wv[2]
                    obuf[buf, q, s] = acc

        def start_write(ci, buf):
            pltpu.async_copy(obuf.at[buf], out_hbm.at[pl.ds(gq0(ci), CQ)],
                             wsem[buf])

        def drain_write(buf):
            pltpu.make_async_copy(obuf.at[buf], out_hbm.at[pl.ds(0, CQ)],
                                  wsem[buf]).wait()

        start_gather(0, 0)
        start_gather(1, 1)

        @pl.loop(0, N_CH // 2 - 1)
        def _(it):
            ci = it * 2
            drain_gather(0)
            compute(0)
            start_gather(ci + 2, 0)
            start_write(ci, 0)
            drain_gather(1)
            compute(1)
            start_gather(ci + 3, 1)
            start_write(ci + 1, 1)
            drain_write(0)
            drain_write(1)

        ci_last = N_CH - 2
        drain_gather(0)
        compute(0)
        start_write(ci_last, 0)
        drain_gather(1)
        compute(1)
        start_write(ci_last + 1, 1)
        drain_write(0)
        drain_write(1)

    return fused_kernel(feat_t, idx2, wtx)


def kernel(query_xyz, input_xyz, input_features):
    ridd = jnp.clip(jnp.arange(H)[None, :] // 2 + jnp.arange(KS0)[:, None] - 5,
                    0, h - 1)
    cid = jnp.clip(jnp.arange(UP_W) // 2 - 10, 0, w - 1)
    upd = input_xyz[:, :, ridd][:, :, :, :, cid]
    upd = upd.transpose(0, 2, 1, 3, 4)

    idx, wts = _knn_search(query_xyz, upd)

    feat_t = input_features.transpose(0, 2, 3, 1).reshape(B * h * w, C)

    wtx = jnp.broadcast_to(wts.reshape(K, NQ).T[:, :, None],
                           (NQ, K, CQ)).reshape(NQ * K * CQ)

    out = _sc_fused(feat_t, idx.reshape(NQ3), wtx)
    return out.reshape(B, H, W, C).transpose(0, 3, 1, 2)


# speedup vs baseline: 1.6193x; 1.6193x over previous
import functools

import jax
import jax.numpy as jnp
from jax import lax
from jax.experimental import pallas as pl
from jax.experimental.pallas import tpu as pltpu
from jax.experimental.pallas import tpu_sc as plsc

B = 2
H, W = 64, 1024
h, w = 32, 512
C = 128
KS0, KS1 = 10, 20
K = 3
DIST2 = 100.0 * 100.0
NQ = B * H * W
NQ3 = K * NQ
UP_H = 84
UP_W = 1064
ROWS_PER_STEP = 8
N_STRIPS = H // ROWS_PER_STEP


def _knn_body(q_ref, up_ref, idx_ref, wt_ref):
    b = pl.program_id(0)
    s = pl.program_id(1)
    shp = (ROWS_PER_STEP, W)
    qx = q_ref[0, 0]
    qy = q_ref[0, 1]
    qz = q_ref[0, 2]

    ri = lax.broadcasted_iota(jnp.int32, shp, 0)
    jj = lax.broadcasted_iota(jnp.int32, shp, 1)
    inf = jnp.full(shp, jnp.inf, jnp.float32)
    zero_i = jnp.zeros(shp, jnp.int32)

    def body(dh, st):
        b0, b1, b2, i0, i1, i2 = st
        u = 8 * s + 2 * dh + ri
        mh = (u >= 10) & (u <= 73)
        cx = up_ref[0, dh, 0]
        cy = up_ref[0, dh, 1]
        cz = up_ref[0, dh, 2]
        for dw in range(KS1):
            t0 = 2 * dw
            dx = cx[:, t0:t0 + W] - qx
            dy = cy[:, t0:t0 + W] - qy
            dz = cz[:, t0:t0 + W] - qz
            d2 = dx * dx + dy * dy + dz * dz
            mw = (jj >= 20 - t0) & (jj <= 1043 - t0)
            d2 = jnp.where(mh & mw, d2, inf)
            ci = jnp.full(shp, dh * KS1 + dw, jnp.int32)
            m2 = d2 < b2
            nb2 = jnp.where(m2, d2, b2)
            ni2 = jnp.where(m2, ci, i2)
            m1 = nb2 < b1
            b2 = jnp.where(m1, b1, nb2)
            i2 = jnp.where(m1, i1, ni2)
            nb1 = jnp.where(m1, nb2, b1)
            ni1 = jnp.where(m1, ni2, i1)
            m0 = nb1 < b0
            b1 = jnp.where(m0, b0, nb1)
            i1 = jnp.where(m0, i0, ni1)
            b0 = jnp.where(m0, nb1, b0)
            i0 = jnp.where(m0, ni1, i0)
        return b0, b1, b2, i0, i1, i2

    st = (inf, inf, inf, zero_i, zero_i, zero_i)
    for dh in range(KS0):
        st = body(dh, st)
    b0, b1, b2, i0, i1, i2 = st

    qn = jnp.sqrt(qx * qx + qy * qy + qz * qz)
    irow_half = (8 * s + ri) >> 1
    jcol_half = jj >> 1
    base_flat = irow_half * w + jcol_half + b * (h * w)

    dists = []
    valids = []
    for bk in (b0, b1, b2):
        vk = bk < DIST2
        dk = jnp.where(vk, jnp.sqrt(bk), qn)
        dk = jnp.maximum(dk, 1e-8)
        dists.append(dk)
        valids.append(vk)
    w0 = 1.0 / dists[0]
    w1 = 1.0 / dists[1]
    w2 = 1.0 / dists[2]
    ws = w0 + w1 + w2
    for k, (ik, wk, vk) in enumerate(((i0, w0, valids[0]),
                                      (i1, w1, valids[1]),
                                      (i2, w2, valids[2]))):
        dh_k = ik // KS1
        dw_k = ik - dh_k * KS1
        flat = base_flat + (dh_k - 5) * w + (dw_k - 10)
        flat = jnp.clip(flat, 0, B * h * w - 1)
        idx_ref[k] = flat
        wt_ref[k] = (wk / ws) * vk.astype(jnp.float32)


def _knn_search(query_xyz, up):
    grid = (B, N_STRIPS)
    out_shape = [
        jax.ShapeDtypeStruct((K, B * H, W), jnp.int32),
        jax.ShapeDtypeStruct((K, B * H, W), jnp.float32),
    ]
    return pl.pallas_call(
        _knn_body,
        grid=grid,
        in_specs=[
            pl.BlockSpec((1, 3, ROWS_PER_STEP, W), lambda b, s: (b, 0, s, 0)),
            pl.BlockSpec((1, KS0, 3, ROWS_PER_STEP, UP_W),
                         lambda b, s: (b, 0, 0, s, 0)),
        ],
        out_specs=[
            pl.BlockSpec((K, ROWS_PER_STEP, W),
                         lambda b, s: (0, b * N_STRIPS + s, 0)),
            pl.BlockSpec((K, ROWS_PER_STEP, W),
                         lambda b, s: (0, b * N_STRIPS + s, 0)),
        ],
        out_shape=out_shape,
        compiler_params=pltpu.CompilerParams(
            dimension_semantics=("parallel", "parallel")),
    )(query_xyz, up)


NW = 32
PER_W = NQ3 // NW
CHUNK = 128


N_CHUNKS = PER_W // CHUNK


def _sc_gather(feat_t, idx_flat):
    mesh = plsc.VectorSubcoreMesh(core_axis_name="c", subcore_axis_name="s")

    @functools.partial(
        pl.kernel,
        mesh=mesh,
        out_type=jax.ShapeDtypeStruct((NQ3, C), jnp.float32),
        scratch_types=[
            pltpu.VMEM((PER_W,), jnp.int32),
            pltpu.VMEM((CHUNK, C), jnp.float32),
            pltpu.VMEM((CHUNK, C), jnp.float32),
            pltpu.SemaphoreType.DMA,
            pltpu.SemaphoreType.DMA,
            pltpu.SemaphoreType.DMA,
            pltpu.SemaphoreType.DMA,
            pltpu.SemaphoreType.DMA,
        ],
    )
    def gather_kernel(feat_hbm, idx_hbm, out_hbm, idx_v, rows0, rows1,
                      isem, g0, g1, w0, w1):
        wid = lax.axis_index("s") * 2 + lax.axis_index("c")
        base = wid * PER_W
        pltpu.async_copy(idx_hbm.at[pl.ds(base, PER_W)], idx_v, isem).wait()

        rows = (rows0, rows1)
        gsem = (g0, g1)
        wsem = (w0, w1)

        def start_gather(ci, buf):
            pltpu.async_copy(
                feat_hbm.at[idx_v.at[pl.ds(ci * CHUNK, CHUNK)]],
                rows[buf], gsem[buf])

        def start_write(ci, buf):
            pltpu.async_copy(
                rows[buf], out_hbm.at[pl.ds(base + ci * CHUNK, CHUNK)],
                wsem[buf])

        def drain(buf, sems):
            pltpu.make_async_copy(rows[buf], out_hbm.at[pl.ds(0, CHUNK)],
                                  sems[buf]).wait()

        start_gather(0, 0)

        @pl.loop(0, N_CHUNKS // 2 - 1)
        def _(it):
            ci = it * 2
            drain(0, gsem)
            start_gather(ci + 1, 1)
            start_write(ci, 0)
            drain(1, gsem)
            drain(0, wsem)
            start_gather(ci + 2, 0)
            start_write(ci + 1, 1)
            drain(1, wsem)

        ci_last = N_CHUNKS - 2
        drain(0, gsem)
        start_gather(ci_last + 1, 1)
        start_write(ci_last, 0)
        drain(1, gsem)
        drain(0, wsem)
        start_write(ci_last + 1, 1)
        drain(1, wsem)

    return gather_kernel(feat_t, idx_flat)


QB = 2048
HW = H * W


def _combine_body(g_ref, w_ref, o_ref):
    w0 = w_ref[:, 0:1]
    w1 = w_ref[:, 1:2]
    w2 = w_ref[:, 2:3]
    acc = g_ref[0] * w0 + g_ref[1] * w1 + g_ref[2] * w2
    o_ref[0] = acc.T


def _combine(g, wt):
    nb = HW // QB

    return pl.pallas_call(
        _combine_body,
        grid=(NQ // QB,),
        in_specs=[
            pl.BlockSpec((K, QB, C), lambda q: (0, q, 0)),
            pl.BlockSpec((QB, K), lambda q: (q, 0)),
        ],
        out_specs=pl.BlockSpec((1, C, QB), lambda q: (q // nb, 0, q % nb)),
        out_shape=jax.ShapeDtypeStruct((B, C, HW), jnp.float32),
        compiler_params=pltpu.CompilerParams(
            dimension_semantics=("parallel",)),
    )(g, wt)


def kernel(query_xyz, input_xyz, input_features):
    ridd = jnp.clip(jnp.arange(H)[None, :] // 2 + jnp.arange(KS0)[:, None] - 5,
                    0, h - 1)
    cid = jnp.clip(jnp.arange(UP_W) // 2 - 10, 0, w - 1)
    upd = input_xyz[:, :, ridd][:, :, :, :, cid]
    upd = upd.transpose(0, 2, 1, 3, 4)

    idx, wts = _knn_search(query_xyz, upd)

    feat_t = input_features.transpose(0, 2, 3, 1).reshape(B * h * w, C)
    g = _sc_gather(feat_t, idx.reshape(NQ3))

    wt = wts.reshape(K, NQ).T
    out = _combine(g.reshape(K, NQ, C), wt)
    return out.reshape(B, C, H, W)


# speedup vs baseline: 1.6750x; 1.0344x over previous
import functools

import jax
import jax.numpy as jnp
from jax import lax
from jax.experimental import pallas as pl
from jax.experimental.pallas import tpu as pltpu
from jax.experimental.pallas import tpu_sc as plsc

B = 2
H, W = 64, 1024
h, w = 32, 512
C = 128
KS0, KS1 = 10, 20
K = 3
DIST2 = 100.0 * 100.0
NQ = B * H * W
NQ3 = K * NQ
UP_H = 84
UP_W = 1064
ROWS_PER_STEP = 8
N_STRIPS = H // ROWS_PER_STEP


def _knn_body(q_ref, up_ref, idx_ref, wt_ref):
    b = pl.program_id(0)
    s = pl.program_id(1)
    shp = (ROWS_PER_STEP, W)
    qx = q_ref[0, 0]
    qy = q_ref[0, 1]
    qz = q_ref[0, 2]

    ri = lax.broadcasted_iota(jnp.int32, shp, 0)
    jj = lax.broadcasted_iota(jnp.int32, shp, 1)
    inf = jnp.full(shp, jnp.inf, jnp.float32)
    zero_i = jnp.zeros(shp, jnp.int32)

    def body(dh, st):
        b0, b1, b2, i0, i1, i2 = st
        u = 8 * s + 2 * dh + ri
        mh = (u >= 10) & (u <= 73)
        cx = up_ref[0, dh, 0]
        cy = up_ref[0, dh, 1]
        cz = up_ref[0, dh, 2]
        for dw in range(KS1):
            t0 = 2 * dw
            dx = cx[:, t0:t0 + W] - qx
            dy = cy[:, t0:t0 + W] - qy
            dz = cz[:, t0:t0 + W] - qz
            d2 = dx * dx + dy * dy + dz * dz
            mw = (jj >= 20 - t0) & (jj <= 1043 - t0)
            d2 = jnp.where(mh & mw, d2, inf)
            ci = jnp.full(shp, dh * KS1 + dw, jnp.int32)
            m2 = d2 < b2
            nb2 = jnp.where(m2, d2, b2)
            ni2 = jnp.where(m2, ci, i2)
            m1 = nb2 < b1
            b2 = jnp.where(m1, b1, nb2)
            i2 = jnp.where(m1, i1, ni2)
            nb1 = jnp.where(m1, nb2, b1)
            ni1 = jnp.where(m1, ni2, i1)
            m0 = nb1 < b0
            b1 = jnp.where(m0, b0, nb1)
            i1 = jnp.where(m0, i0, ni1)
            b0 = jnp.where(m0, nb1, b0)
            i0 = jnp.where(m0, ni1, i0)
        return b0, b1, b2, i0, i1, i2

    st = (inf, inf, inf, zero_i, zero_i, zero_i)
    for dh in range(KS0):
        st = body(dh, st)
    b0, b1, b2, i0, i1, i2 = st

    qn = jnp.sqrt(qx * qx + qy * qy + qz * qz)
    irow_half = (8 * s + ri) >> 1
    jcol_half = jj >> 1
    base_flat = irow_half * w + jcol_half + b * (h * w)

    dists = []
    valids = []
    for bk in (b0, b1, b2):
        vk = bk < DIST2
        dk = jnp.where(vk, jnp.sqrt(bk), qn)
        dk = jnp.maximum(dk, 1e-8)
        dists.append(dk)
        valids.append(vk)
    w0 = 1.0 / dists[0]
    w1 = 1.0 / dists[1]
    w2 = 1.0 / dists[2]
    ws = w0 + w1 + w2
    for k, (ik, wk, vk) in enumerate(((i0, w0, valids[0]),
                                      (i1, w1, valids[1]),
                                      (i2, w2, valids[2]))):
        dh_k = ik // KS1
        dw_k = ik - dh_k * KS1
        flat = base_flat + (dh_k - 5) * w + (dw_k - 10)
        flat = jnp.clip(flat, 0, B * h * w - 1)
        idx_ref[k] = flat
        wt_ref[k] = (wk / ws) * vk.astype(jnp.float32)


def _knn_search(query_xyz, up):
    grid = (B, N_STRIPS)
    out_shape = [
        jax.ShapeDtypeStruct((K, B * H, W), jnp.int32),
        jax.ShapeDtypeStruct((K, B * H, W), jnp.float32),
    ]
    return pl.pallas_call(
        _knn_body,
        grid=grid,
        in_specs=[
            pl.BlockSpec((1, 3, ROWS_PER_STEP, W), lambda b, s: (b, 0, s, 0)),
            pl.BlockSpec((1, KS0, 3, ROWS_PER_STEP, UP_W),
                         lambda b, s: (b, 0, 0, s, 0)),
        ],
        out_specs=[
            pl.BlockSpec((K, ROWS_PER_STEP, W),
                         lambda b, s: (0, b * N_STRIPS + s, 0)),
            pl.BlockSpec((K, ROWS_PER_STEP, W),
                         lambda b, s: (0, b * N_STRIPS + s, 0)),
        ],
        out_shape=out_shape,
        compiler_params=pltpu.CompilerParams(
            dimension_semantics=("parallel", "parallel")),
    )(query_xyz, up)


def _feat_t_body(f_ref, o_ref):
    for r in range(8):
        o_ref[r] = f_ref[0, :, r, :].T


def _feat_transpose(input_features):
    return pl.pallas_call(
        _feat_t_body,
        grid=(B, h // 8),
        in_specs=[pl.BlockSpec((1, C, 8, w), lambda b, y: (b, 0, y, 0))],
        out_specs=pl.BlockSpec((8, w, C), lambda b, y: (b * (h // 8) + y, 0, 0)),
        out_shape=jax.ShapeDtypeStruct((B * h, w, C), jnp.float32),
        compiler_params=pltpu.CompilerParams(
            dimension_semantics=("parallel", "parallel")),
    )(input_features).reshape(B * h * w, C)


NW = 32
PER_W = NQ3 // NW
CHUNK = 128


N_CHUNKS = PER_W // CHUNK


def _sc_gather(feat_t, idx_flat):
    mesh = plsc.VectorSubcoreMesh(core_axis_name="c", subcore_axis_name="s")

    @functools.partial(
        pl.kernel,
        mesh=mesh,
        out_type=jax.ShapeDtypeStruct((NQ3, C), jnp.float32),
        scratch_types=[
            pltpu.VMEM((PER_W,), jnp.int32),
            pltpu.VMEM((CHUNK, C), jnp.float32),
            pltpu.VMEM((CHUNK, C), jnp.float32),
            pltpu.SemaphoreType.DMA,
            pltpu.SemaphoreType.DMA,
            pltpu.SemaphoreType.DMA,
            pltpu.SemaphoreType.DMA,
            pltpu.SemaphoreType.DMA,
        ],
    )
    def gather_kernel(feat_hbm, idx_hbm, out_hbm, idx_v, rows0, rows1,
                      isem, g0, g1, w0, w1):
        wid = lax.axis_index("s") * 2 + lax.axis_index("c")
        base = wid * PER_W
        pltpu.async_copy(idx_hbm.at[pl.ds(base, PER_W)], idx_v, isem).wait()

        rows = (rows0, rows1)
        gsem = (g0, g1)
        wsem = (w0, w1)

        def start_gather(ci, buf):
            pltpu.async_copy(
                feat_hbm.at[idx_v.at[pl.ds(ci * CHUNK, CHUNK)]],
                rows[buf], gsem[buf])

        def start_write(ci, buf):
            pltpu.async_copy(
                rows[buf], out_hbm.at[pl.ds(base + ci * CHUNK, CHUNK)],
                wsem[buf])

        def drain(buf, sems):
            pltpu.make_async_copy(rows[buf], out_hbm.at[pl.ds(0, CHUNK)],
                                  sems[buf]).wait()

        start_gather(0, 0)

        @pl.loop(0, N_CHUNKS // 2 - 1)
        def _(it):
            ci = it * 2
            drain(0, gsem)
            start_gather(ci + 1, 1)
            start_write(ci, 0)
            drain(1, gsem)
            drain(0, wsem)
            start_gather(ci + 2, 0)
            start_write(ci + 1, 1)
            drain(1, wsem)

        ci_last = N_CHUNKS - 2
        drain(0, gsem)
        start_gather(ci_last + 1, 1)
        start_write(ci_last, 0)
        drain(1, gsem)
        drain(0, wsem)
        start_write(ci_last + 1, 1)
        drain(1, wsem)

    return gather_kernel(feat_t, idx_flat)


QB = 2048
HW = H * W


def _combine_body(g_ref, w_ref, o_ref):
    w0 = w_ref[:, 0:1]
    w1 = w_ref[:, 1:2]
    w2 = w_ref[:, 2:3]
    acc = g_ref[0] * w0 + g_ref[1] * w1 + g_ref[2] * w2
    o_ref[0] = acc.T


def _combine(g, wt):
    nb = HW // QB

    return pl.pallas_call(
        _combine_body,
        grid=(NQ // QB,),
        in_specs=[
            pl.BlockSpec((K, QB, C), lambda q: (0, q, 0)),
            pl.BlockSpec((QB, K), lambda q: (q, 0)),
        ],
        out_specs=pl.BlockSpec((1, C, QB), lambda q: (q // nb, 0, q % nb)),
        out_shape=jax.ShapeDtypeStruct((B, C, HW), jnp.float32),
        compiler_params=pltpu.CompilerParams(
            dimension_semantics=("parallel",)),
    )(g, wt)


def kernel(query_xyz, input_xyz, input_features):
    ridd = jnp.clip(jnp.arange(H)[None, :] // 2 + jnp.arange(KS0)[:, None] - 5,
                    0, h - 1)
    cid = jnp.clip(jnp.arange(UP_W) // 2 - 10, 0, w - 1)
    upd = input_xyz[:, :, ridd][:, :, :, :, cid]
    upd = upd.transpose(0, 2, 1, 3, 4)

    idx, wts = _knn_search(query_xyz, upd)

    feat_t = _feat_transpose(input_features)
    g = _sc_gather(feat_t, idx.reshape(NQ3))

    wt = wts.reshape(K, NQ).T
    out = _combine(g.reshape(K, NQ, C), wt)
    return out.reshape(B, C, H, W)
